# baseline (speedup 1.0000x reference)
import functools

import jax
import jax.numpy as jnp
from jax import lax
from jax.experimental import pallas as pl
from jax.experimental.pallas import tpu as pltpu
from jax.experimental.pallas import tpu_sc as plsc

_INFO = plsc.get_sparse_core_info()
_NC = _INFO.num_cores
_NS = _INFO.num_subcores
_NW = _NC * _NS
_CHUNK = 64
_TABLE_PAD = 1024


@functools.partial(jax.jit, static_argnames=())
def _embed_lookup(labels2d, table):
    n_rows, chunk = labels2d.shape
    v, d = table.shape
    b = n_rows * chunk
    b_per_w = b // _NW
    nch = b_per_w // chunk

    mesh = plsc.VectorSubcoreMesh(core_axis_name="c", subcore_axis_name="s")

    @functools.partial(
        pl.kernel,
        mesh=mesh,
        out_type=jax.ShapeDtypeStruct((b, d), jnp.float32),
        scratch_types=[
            pltpu.VMEM((nch, chunk), jnp.int32),
            pltpu.VMEM((b_per_w, d), jnp.float32),
            pltpu.VMEM_SHARED((v, d), jnp.float32),
        ]
        + [pltpu.SemaphoreType.DMA] * nch
        + [pltpu.SemaphoreType.DMA, pltpu.SemaphoreType.DMA],
    )
    def run(labels_hbm, table_hbm, out_hbm, idx_v, rows_v, table_sh, *sems):
        gather_sems, store_sem, lbl_sem = sems[:nch], sems[nch], sems[nch + 1]
        sid = lax.axis_index("s")
        wid = sid * _NC + lax.axis_index("c")
        row_base = wid * nch
        base = wid * b_per_w
        lbl = pltpu.async_copy(labels_hbm.at[pl.ds(row_base, nch)], idx_v, lbl_sem)
        rows_per_tile = v // _NS
        pltpu.sync_copy(
            table_hbm.at[pl.ds(sid * rows_per_tile, rows_per_tile)],
            table_sh.at[pl.ds(sid * rows_per_tile, rows_per_tile)],
        )
        plsc.subcore_barrier()
        lbl.wait()
        gathers = [
            pltpu.async_copy(
                table_sh.at[idx_v.at[j]],
                rows_v.at[pl.ds(j * chunk, chunk)],
                gather_sems[j],
            )
            for j in range(nch)
        ]
        stores = []
        for j in range(nch):
            gathers[j].wait()
            stores.append(
                pltpu.async_copy(
                    rows_v.at[pl.ds(j * chunk, chunk)],
                    out_hbm.at[pl.ds(base + j * chunk, chunk)],
                    store_sem,
                )
            )
        for s in stores:
            s.wait()

    return run(labels2d, table)


def kernel(labels, table):
    (b,) = labels.shape
    labels2d = labels.astype(jnp.int32).reshape(b // _CHUNK, _CHUNK)
    v, d = table.shape
    table_p = jnp.pad(table, ((0, _TABLE_PAD - v), (0, 0)))
    return _embed_lookup(labels2d, table_p)


# speedup vs baseline: 1.0162x; 1.0162x over previous
import functools

import jax
import jax.numpy as jnp
from jax import lax
from jax.experimental import pallas as pl
from jax.experimental.pallas import tpu as pltpu
from jax.experimental.pallas import tpu_sc as plsc

_INFO = plsc.get_sparse_core_info()
_NC = _INFO.num_cores
_NS = _INFO.num_subcores
_NW = _NC * _NS
_CHUNK = 128
_TABLE_PAD = 1024


@functools.partial(jax.jit, static_argnames=())
def _embed_lookup(labels2d, table):
    n_rows, chunk = labels2d.shape
    v, d = table.shape
    b = n_rows * chunk
    b_per_w = b // _NW
    nch = b_per_w // chunk

    mesh = plsc.VectorSubcoreMesh(core_axis_name="c", subcore_axis_name="s")

    @functools.partial(
        pl.kernel,
        mesh=mesh,
        out_type=jax.ShapeDtypeStruct((b, d), jnp.float32),
        scratch_types=[
            pltpu.VMEM((nch, chunk), jnp.int32),
            pltpu.VMEM((b_per_w, d), jnp.float32),
            pltpu.VMEM_SHARED((v, d), jnp.float32),
        ]
        + [pltpu.SemaphoreType.DMA] * nch
        + [pltpu.SemaphoreType.DMA, pltpu.SemaphoreType.DMA],
    )
    def run(labels_hbm, table_hbm, out_hbm, idx_v, rows_v, table_sh, *sems):
        gather_sems, store_sem, lbl_sem = sems[:nch], sems[nch], sems[nch + 1]
        sid = lax.axis_index("s")
        wid = sid * _NC + lax.axis_index("c")
        row_base = wid * nch
        base = wid * b_per_w
        lbl = pltpu.async_copy(labels_hbm.at[pl.ds(row_base, nch)], idx_v, lbl_sem)
        rows_per_tile = v // _NS
        pltpu.sync_copy(
            table_hbm.at[pl.ds(sid * rows_per_tile, rows_per_tile)],
            table_sh.at[pl.ds(sid * rows_per_tile, rows_per_tile)],
        )
        plsc.subcore_barrier()
        lbl.wait()
        gathers = [
            pltpu.async_copy(
                table_sh.at[idx_v.at[j]],
                rows_v.at[pl.ds(j * chunk, chunk)],
                gather_sems[j],
            )
            for j in range(nch)
        ]
        stores = []
        for j in range(nch):
            gathers[j].wait()
            stores.append(
                pltpu.async_copy(
                    rows_v.at[pl.ds(j * chunk, chunk)],
                    out_hbm.at[pl.ds(base + j * chunk, chunk)],
                    store_sem,
                )
            )
        for s in stores:
            s.wait()

    return run(labels2d, table)


def kernel(labels, table):
    (b,) = labels.shape
    labels2d = labels.astype(jnp.int32).reshape(b // _CHUNK, _CHUNK)
    v, d = table.shape
    table_p = jnp.pad(table, ((0, _TABLE_PAD - v), (0, 0)))
    return _embed_lookup(labels2d, table_p)


# speedup vs baseline: 1.0243x; 1.0080x over previous
import functools

import jax
import jax.numpy as jnp
from jax import lax
from jax.experimental import pallas as pl
from jax.experimental.pallas import tpu as pltpu
from jax.experimental.pallas import tpu_sc as plsc

_INFO = plsc.get_sparse_core_info()
_NC = _INFO.num_cores
_NS = _INFO.num_subcores
_NW = _NC * _NS
_CHUNK = 128


@functools.partial(jax.jit, static_argnames=())
def _embed_lookup(labels2d, table):
    n_rows, chunk = labels2d.shape
    v, d = table.shape
    b = n_rows * chunk
    b_per_w = b // _NW
    nch = b_per_w // chunk

    mesh = plsc.VectorSubcoreMesh(core_axis_name="c", subcore_axis_name="s")

    @functools.partial(
        pl.kernel,
        mesh=mesh,
        out_type=jax.ShapeDtypeStruct((b, d), jnp.float32),
        scratch_types=[
            pltpu.VMEM((nch, chunk), jnp.int32),
            pltpu.VMEM((b_per_w, d), jnp.float32),
            pltpu.VMEM_SHARED((v, d), jnp.float32),
        ]
        + [pltpu.SemaphoreType.DMA] * nch
        + [pltpu.SemaphoreType.DMA, pltpu.SemaphoreType.DMA],
    )
    def run(labels_hbm, table_hbm, out_hbm, idx_v, rows_v, table_sh, *sems):
        gather_sems, store_sem, lbl_sem = sems[:nch], sems[nch], sems[nch + 1]
        sid = lax.axis_index("s")
        wid = sid * _NC + lax.axis_index("c")
        row_base = wid * nch
        base = wid * b_per_w
        lbl = pltpu.async_copy(labels_hbm.at[pl.ds(row_base, nch)], idx_v, lbl_sem)
        @pl.when(sid == 0)
        def _():
            pltpu.sync_copy(table_hbm, table_sh)

        plsc.subcore_barrier()
        lbl.wait()
        gathers = [
            pltpu.async_copy(
                table_sh.at[idx_v.at[j]],
                rows_v.at[pl.ds(j * chunk, chunk)],
                gather_sems[j],
            )
            for j in range(nch)
        ]
        stores = []
        for j in range(nch):
            gathers[j].wait()
            stores.append(
                pltpu.async_copy(
                    rows_v.at[pl.ds(j * chunk, chunk)],
                    out_hbm.at[pl.ds(base + j * chunk, chunk)],
                    store_sem,
                )
            )
        for s in stores:
            s.wait()

    return run(labels2d, table)


def kernel(labels, table):
    (b,) = labels.shape
    labels2d = labels.astype(jnp.int32).reshape(b // _CHUNK, _CHUNK)
    return _embed_lookup(labels2d, table)


# speedup vs baseline: 1.0283x; 1.0039x over previous
import functools

import jax
import jax.numpy as jnp
from jax import lax
from jax.experimental import pallas as pl
from jax.experimental.pallas import tpu as pltpu
from jax.experimental.pallas import tpu_sc as plsc

_INFO = plsc.get_sparse_core_info()
_NC = _INFO.num_cores
_NS = _INFO.num_subcores
_NW = _NC * _NS
_CHUNK = 128


@functools.partial(jax.jit, static_argnames=())
def _embed_lookup(labels2d, table):
    n_rows, chunk = labels2d.shape
    v, d = table.shape
    b = n_rows * chunk
    b_per_w = b // _NW
    nch = b_per_w // chunk

    mesh = plsc.VectorSubcoreMesh(core_axis_name="c", subcore_axis_name="s")

    @functools.partial(
        pl.kernel,
        mesh=mesh,
        out_type=jax.ShapeDtypeStruct((b, d), jnp.float32),
        scratch_types=[
            pltpu.VMEM((nch, chunk), jnp.int32),
            pltpu.VMEM((b_per_w, d), jnp.float32),
            pltpu.VMEM_SHARED((v, d), jnp.float32),
        ]
        + [pltpu.SemaphoreType.DMA] * nch
        + [pltpu.SemaphoreType.DMA, pltpu.SemaphoreType.DMA],
    )
    def run(labels_hbm, table_hbm, out_hbm, idx_v, rows_v, table_sh, *sems):
        gather_sems, store_sem, lbl_sem = sems[:nch], sems[nch], sems[nch + 1]
        sid = lax.axis_index("s")
        wid = sid * _NC + lax.axis_index("c")
        row_base = wid * nch
        base = wid * b_per_w
        lbl = pltpu.async_copy(labels_hbm.at[pl.ds(row_base, nch)], idx_v, lbl_sem)
        @pl.when(sid == 0)
        def _():
            pltpu.sync_copy(table_hbm, table_sh)

        lbl.wait()
        gathers = [
            pltpu.async_copy(
                table_hbm.at[idx_v.at[0]],
                rows_v.at[pl.ds(0, chunk)],
                gather_sems[0],
            )
        ]
        plsc.subcore_barrier()
        gathers += [
            pltpu.async_copy(
                table_sh.at[idx_v.at[j]],
                rows_v.at[pl.ds(j * chunk, chunk)],
                gather_sems[j],
            )
            for j in range(1, nch)
        ]
        stores = []
        for j in range(nch):
            gathers[j].wait()
            stores.append(
                pltpu.async_copy(
                    rows_v.at[pl.ds(j * chunk, chunk)],
                    out_hbm.at[pl.ds(base + j * chunk, chunk)],
                    store_sem,
                )
            )
        for s in stores:
            s.wait()

    return run(labels2d, table)


def kernel(labels, table):
    (b,) = labels.shape
    labels2d = labels.astype(jnp.int32).reshape(b // _CHUNK, _CHUNK)
    return _embed_lookup(labels2d, table)
